# Initial kernel scaffold; baseline (speedup 1.0000x reference)
import jax
import jax.numpy as jnp
from jax.experimental import pallas as pl


def kernel(x, edge_index, edge_type, cg_norm, index1, index2, f2, tar_rel, W_w, b_w, W_w2, b_w2, attn_rel_emb, self_loop_weight, zero_path, W_line, b_line):
    raise NotImplementedError("write your pallas kernel here")


# speedup vs baseline: 3.1351x; 3.1351x over previous
import functools

import jax
import jax.numpy as jnp
from jax import lax
from jax.experimental import pallas as pl
from jax.experimental.pallas import tpu as pltpu
from jax.experimental.pallas import tpu_sc as plsc

N = 10000
E = 160000
D = 128
R = 10
B = 16
NPG = N // B

NC = 2
NS = 16
NW = NC * NS
K = 128
EPT = 5120
EPAD = NW * EPT
NCHUNK = EPT // K

F32 = jnp.float32


def _wprep_body(w2_ref, attn_ref, bw2_ref, wline_ref, slw_ref,
                ab_ref, c2_ref, v_ref, sll_ref):
    w2a = w2_ref[0:D, :]
    w2b = w2_ref[D:2 * D, :]
    w2c = w2_ref[2 * D:3 * D, :]
    wline = wline_ref[...]
    dot = functools.partial(jnp.dot, preferred_element_type=F32)
    ab_ref[...] = dot(w2a - w2b, wline)
    c2_ref[...] = dot(w2c, wline)
    v_ref[...] = dot(dot(attn_ref[...], w2a + w2b) + bw2_ref[...], wline)
    sll_ref[...] = dot(slw_ref[...], wline)


def _wprep(W_w2, attn, b_w2, W_line, slw):
    return pl.pallas_call(
        _wprep_body,
        out_shape=(
            jax.ShapeDtypeStruct((D, D), F32),
            jax.ShapeDtypeStruct((D, D), F32),
            jax.ShapeDtypeStruct((R, D), F32),
            jax.ShapeDtypeStruct((D, D), F32),
        ),
    )(W_w2, attn, b_w2.reshape(1, D), W_line, slw)


BN = 1000


def _tables_body(x_ref, ww_ref, bw_ref, ab_ref, c2_ref, v_ref, sll_ref,
                 bline_ref, attn_ref, m_ref, xsl_ref):
    dot = functools.partial(jnp.dot, preferred_element_type=F32)
    x = x_ref[...]
    w1 = dot(x, ww_ref[...]) + bw_ref[...]
    wab = dot(w1, ab_ref[...])
    c2 = c2_ref[...]
    xsl_ref[...] = dot(x, sll_ref[...]) + bline_ref[...]
    for t in range(R):
        at = attn_ref[t:t + 1, :]
        m_ref[t] = dot(w1 * at, c2) + wab + v_ref[t:t + 1, :]


def _tables(x, W_w, b_w, AB, C2, V, SLL, b_line, attn):
    grid = N // BN
    mbig, xsl = pl.pallas_call(
        _tables_body,
        grid=(grid,),
        in_specs=[
            pl.BlockSpec((BN, D), lambda i: (i, 0)),
            pl.BlockSpec((D, D), lambda i: (0, 0)),
            pl.BlockSpec((1, D), lambda i: (0, 0)),
            pl.BlockSpec((D, D), lambda i: (0, 0)),
            pl.BlockSpec((D, D), lambda i: (0, 0)),
            pl.BlockSpec((R, D), lambda i: (0, 0)),
            pl.BlockSpec((D, D), lambda i: (0, 0)),
            pl.BlockSpec((1, D), lambda i: (0, 0)),
            pl.BlockSpec((R, D), lambda i: (0, 0)),
        ],
        out_specs=(
            pl.BlockSpec((R, BN, D), lambda i: (0, i, 0)),
            pl.BlockSpec((BN, D), lambda i: (i, 0)),
        ),
        out_shape=(
            jax.ShapeDtypeStruct((R, N, D), F32),
            jax.ShapeDtypeStruct((N, D), F32),
        ),
    )(x, W_w, b_w.reshape(1, D), AB, C2, V, SLL, b_line.reshape(1, D), attn)
    return mbig.reshape(R * N, D), xsl


def _sc_edges_body(mbig_hbm, et_hbm, src_hbm, dst_hbm, cg_hbm, out_hbm,
                   et_v, src_v, dst_v, cg_v, idx_v, rows_v, agg, sem):
    cid = lax.axis_index("c")
    sid = lax.axis_index("s")
    base = (cid * NS + sid) * EPT

    zero16 = jnp.zeros((16,), F32)

    def zero_body(j, carry):
        for k in range(8):
            rows_v[j, pl.ds(k * 16, 16)] = zero16
        return carry

    lax.fori_loop(0, K, zero_body, 0)
    for i in range(5):
        pltpu.sync_copy(rows_v.at[pl.ds(0, 125)],
                        agg.at[pl.ds(sid * 625 + i * 125, 125)])
    plsc.subcore_barrier()

    def chunk_body(c, carry):
        off = pl.multiple_of(base + c * K, K)
        pltpu.sync_copy(et_hbm.at[pl.ds(off, K)], et_v)
        pltpu.sync_copy(src_hbm.at[pl.ds(off, K)], src_v)
        pltpu.sync_copy(dst_hbm.at[pl.ds(off, K)], dst_v)
        pltpu.sync_copy(cg_hbm.at[pl.ds(off, K)], cg_v.at[pl.ds(0, K)])
        for i in range(8):
            sl = pl.ds(i * 16, 16)
            idx_v[sl] = et_v[sl] * N + src_v[sl]
        pltpu.async_copy(mbig_hbm.at[idx_v], rows_v, sem).wait()

        def scale_body(j, carry2):
            cgj = cg_v[pl.ds(j, 16)][0]
            for k in range(8):
                sl2 = pl.ds(k * 16, 16)
                rows_v[j, sl2] = rows_v[j, sl2] * cgj
            return carry2

        lax.fori_loop(0, K, scale_body, 0)
        pltpu.sync_copy(rows_v, agg.at[dst_v], add=True)
        return carry

    lax.fori_loop(0, NCHUNK, chunk_body, 0)
    plsc.subcore_barrier()
    row = pl.multiple_of(sid * 624, 8)
    orow = pl.multiple_of(cid * N + sid * 624, 8)
    pltpu.sync_copy(agg.at[pl.ds(row, 624)], out_hbm.at[pl.ds(orow, 624)])

    @pl.when(sid == NS - 1)
    def _tail():
        t0 = pl.multiple_of(16 * 624, 8)
        to = pl.multiple_of(cid * N + 16 * 624, 8)
        pltpu.sync_copy(agg.at[pl.ds(t0, N - 16 * 624)],
                        out_hbm.at[pl.ds(to, N - 16 * 624)])


def _sc_edges(mbig, etp, srcp, dstp, cgp):
    mesh = plsc.VectorSubcoreMesh(core_axis_name="c", subcore_axis_name="s",
                                  num_cores=NC, num_subcores=NS)
    return pl.kernel(
        _sc_edges_body,
        out_type=jax.ShapeDtypeStruct((NC * N, D), F32),
        mesh=mesh,
        scratch_types=[
            pltpu.VMEM((K,), jnp.int32),
            pltpu.VMEM((K,), jnp.int32),
            pltpu.VMEM((K,), jnp.int32),
            pltpu.VMEM((K + 16,), F32),
            pltpu.VMEM((K,), jnp.int32),
            pltpu.VMEM((K, D), F32),
            pltpu.VMEM_SHARED((N, D), F32),
            pltpu.SemaphoreType.DMA,
        ],
    )(mbig, etp, srcp, dstp, cgp)


def _epi_body(p_ref, xsl_ref, i1_ref, i2_ref, f2_ref, tar_ref, zp_ref,
              nr_ref, tg_ref, pa_ref):
    h = p_ref[0, 0] + p_ref[1, 0] + xsl_ref[0]
    nf = jnp.maximum(h, 0.0)
    scat = jnp.where(i2_ref[0] != 0, f2_ref[0] + 1, R + 1)
    iota_j = lax.broadcasted_iota(jnp.int32, (16, NPG), 1)
    rid = lax.broadcasted_iota(jnp.int32, (16, NPG), 0)
    m = (scat == rid) & (rid < R + 1)
    jsel = jnp.max(jnp.where(m, iota_j, -1), axis=1, keepdims=True)
    jt = jnp.min(jnp.where(tar_ref[0] == 1, iota_j[0:1, :], N))
    rr = jnp.max(rid, axis=1, keepdims=True)
    jfin = jnp.where(rr == R + 1, jt, jsel)
    oh = (iota_j == jfin).astype(F32)
    sel = jnp.dot(oh, nf, preferred_element_type=F32)
    nr_ref[0] = sel[0:R + 1, :]
    tg = sel[R + 1:R + 2, :]
    tg_ref[0] = tg
    logits = jnp.dot(nf, tg.T, preferred_element_type=F32)
    pm = i1_ref[0] == 1
    mx = jnp.max(jnp.where(pm, logits, -1e30))
    ex = jnp.where(pm, jnp.exp(logits - mx), 0.0)
    s = jnp.sum(ex)
    pooled = jnp.sum(nf * ex, axis=0, keepdims=True) / jnp.maximum(s, 1e-30)
    pa_ref[0] = jnp.where(s > 0.0, pooled, zp_ref[...])


def _epilogue(parts, xsl, index1, index2, f2, tar_rel, zero_path):
    p4 = parts.reshape(NC, B, NPG, D)
    xsl3 = xsl.reshape(B, NPG, D)
    i1 = index1.reshape(B, NPG, 1)
    i2 = index2.reshape(B, 1, NPG)
    f2r = f2.reshape(B, 1, NPG)
    tr = tar_rel.reshape(B, 1, NPG)
    nr, tg, pa = pl.pallas_call(
        _epi_body,
        grid=(B,),
        in_specs=[
            pl.BlockSpec((NC, 1, NPG, D), lambda i: (0, i, 0, 0)),
            pl.BlockSpec((1, NPG, D), lambda i: (i, 0, 0)),
            pl.BlockSpec((1, NPG, 1), lambda i: (i, 0, 0)),
            pl.BlockSpec((1, 1, NPG), lambda i: (i, 0, 0)),
            pl.BlockSpec((1, 1, NPG), lambda i: (i, 0, 0)),
            pl.BlockSpec((1, 1, NPG), lambda i: (i, 0, 0)),
            pl.BlockSpec((1, D), lambda i: (0, 0)),
        ],
        out_specs=(
            pl.BlockSpec((1, R + 1, D), lambda i: (i, 0, 0)),
            pl.BlockSpec((1, 1, D), lambda i: (i, 0, 0)),
            pl.BlockSpec((1, 1, D), lambda i: (i, 0, 0)),
        ),
        out_shape=(
            jax.ShapeDtypeStruct((B, R + 1, D), F32),
            jax.ShapeDtypeStruct((B, 1, D), F32),
            jax.ShapeDtypeStruct((B, 1, D), F32),
        ),
    )(p4, xsl3, i1, i2, f2r, tr, zero_path)
    return nr, tg.reshape(B, D), pa.reshape(B, D)


def kernel(x, edge_index, edge_type, cg_norm, index1, index2, f2, tar_rel,
           W_w, b_w, W_w2, b_w2, attn_rel_emb, self_loop_weight, zero_path,
           W_line, b_line):
    AB, C2, V, SLL = _wprep(W_w2, attn_rel_emb, b_w2, W_line,
                            self_loop_weight)
    mbig, xsl = _tables(x, W_w, b_w, AB, C2, V, SLL, b_line, attn_rel_emb)

    pad = EPAD - E
    zi = jnp.zeros((pad,), jnp.int32)
    etp = jnp.concatenate([edge_type, zi])
    srcp = jnp.concatenate([edge_index[0], zi])
    dstp = jnp.concatenate([edge_index[1], zi])
    cgp = jnp.concatenate([cg_norm, jnp.zeros((pad,), F32)])

    parts = _sc_edges(mbig, etp, srcp, dstp, cgp)
    return _epilogue(parts, xsl, index1, index2, f2, tar_rel, zero_path)


# speedup vs baseline: 3.3913x; 1.0817x over previous
import functools

import jax
import jax.numpy as jnp
from jax import lax
from jax.experimental import pallas as pl
from jax.experimental.pallas import tpu as pltpu
from jax.experimental.pallas import tpu_sc as plsc

N = 10000
E = 160000
D = 128
R = 10
B = 16
NPG = N // B

NC = 2
NS = 16
NW = NC * NS
K = 128
EPT = 5120
EPAD = NW * EPT
NCHUNK = EPT // K

F32 = jnp.float32


def _wprep_body(w2_ref, attn_ref, bw2_ref, wline_ref, slw_ref,
                ab_ref, c2_ref, v_ref, sll_ref):
    w2a = w2_ref[0:D, :]
    w2b = w2_ref[D:2 * D, :]
    w2c = w2_ref[2 * D:3 * D, :]
    wline = wline_ref[...]
    dot = functools.partial(jnp.dot, preferred_element_type=F32,
                            precision=lax.Precision.HIGHEST)
    ab_ref[...] = dot(w2a - w2b, wline)
    c2_ref[...] = dot(w2c, wline)
    v_ref[...] = dot(dot(attn_ref[...], w2a + w2b) + bw2_ref[...], wline)
    sll_ref[...] = dot(slw_ref[...], wline)


def _wprep(W_w2, attn, b_w2, W_line, slw):
    return pl.pallas_call(
        _wprep_body,
        out_shape=(
            jax.ShapeDtypeStruct((D, D), F32),
            jax.ShapeDtypeStruct((D, D), F32),
            jax.ShapeDtypeStruct((R, D), F32),
            jax.ShapeDtypeStruct((D, D), F32),
        ),
    )(W_w2, attn, b_w2.reshape(1, D), W_line, slw)


BN = 1000


def _tables_body(x_ref, ww_ref, bw_ref, ab_ref, c2_ref, v_ref, sll_ref,
                 bline_ref, attn_ref, m_ref, xsl_ref):
    dot = functools.partial(jnp.dot, preferred_element_type=F32,
                            precision=lax.Precision.HIGHEST)
    x = x_ref[...]
    w1 = dot(x, ww_ref[...]) + bw_ref[...]
    wab = dot(w1, ab_ref[...])
    c2 = c2_ref[...]
    xsl_ref[...] = dot(x, sll_ref[...]) + bline_ref[...]
    for t in range(R):
        at = attn_ref[t:t + 1, :]
        m_ref[t] = dot(w1 * at, c2) + wab + v_ref[t:t + 1, :]


def _tables(x, W_w, b_w, AB, C2, V, SLL, b_line, attn):
    grid = N // BN
    mbig, xsl = pl.pallas_call(
        _tables_body,
        grid=(grid,),
        in_specs=[
            pl.BlockSpec((BN, D), lambda i: (i, 0)),
            pl.BlockSpec((D, D), lambda i: (0, 0)),
            pl.BlockSpec((1, D), lambda i: (0, 0)),
            pl.BlockSpec((D, D), lambda i: (0, 0)),
            pl.BlockSpec((D, D), lambda i: (0, 0)),
            pl.BlockSpec((R, D), lambda i: (0, 0)),
            pl.BlockSpec((D, D), lambda i: (0, 0)),
            pl.BlockSpec((1, D), lambda i: (0, 0)),
            pl.BlockSpec((R, D), lambda i: (0, 0)),
        ],
        out_specs=(
            pl.BlockSpec((R, BN, D), lambda i: (0, i, 0)),
            pl.BlockSpec((BN, D), lambda i: (i, 0)),
        ),
        out_shape=(
            jax.ShapeDtypeStruct((R, N, D), F32),
            jax.ShapeDtypeStruct((N, D), F32),
        ),
    )(x, W_w, b_w.reshape(1, D), AB, C2, V, SLL, b_line.reshape(1, D), attn)
    return mbig.reshape(R * N, D), xsl


def _sc_edges_body(mbig_hbm, pk_hbm, cg_hbm, out_hbm,
                   pk0, pk1, cgv0, cgv1, idx0, idx1, dst0, dst1,
                   rows0, rows1, agg, sem0, sem1):
    cid = lax.axis_index("c")
    sid = lax.axis_index("s")
    gbase = (cid * NS + sid) * NCHUNK

    pk = (pk0, pk1)
    cgv = (cgv0, cgv1)
    idx = (idx0, idx1)
    dst = (dst0, dst1)
    rows = (rows0, rows1)
    sem = (sem0, sem1)

    zero16 = jnp.zeros((16,), F32)

    def zero_body(j, carry):
        for k in range(8):
            rows0[j, pl.ds(k * 16, 16)] = zero16
        return carry

    lax.fori_loop(0, K, zero_body, 0)
    for i in range(5):
        pltpu.sync_copy(rows0.at[pl.ds(0, 125)],
                        agg.at[pl.ds(sid * 625 + i * 125, 125)])
    plsc.subcore_barrier()

    def stage(c, b):
        off = pl.multiple_of(c * (3 * K), 3 * K)
        pltpu.sync_copy(pk_hbm.at[pl.ds(off, 3 * K)], pk[b])
        offc = pl.multiple_of(c * K, K)
        pltpu.sync_copy(cg_hbm.at[pl.ds(offc, K)], cgv[b])
        for i in range(8):
            sl = pl.ds(i * 16, 16)
            idx[b][sl] = pk[b][pl.ds(i * 16, 16)] * N + pk[b][pl.ds(K + i * 16, 16)]
            dst[b][sl] = pk[b][pl.ds(2 * K + i * 16, 16)]
        pltpu.make_async_copy(mbig_hbm.at[idx[b]], rows[b], sem[b]).start()

    def consume(b):
        pltpu.make_async_copy(mbig_hbm.at[idx[b]], rows[b], sem[b]).wait()

        def scale_group(g, carry):
            cg16 = cgv[b][pl.ds(g * 16, 16)]
            for l in range(16):
                c = cg16[l]
                j = g * 16 + l
                for k in range(8):
                    sl2 = pl.ds(k * 16, 16)
                    rows[b][j, sl2] = rows[b][j, sl2] * c
            return carry

        lax.fori_loop(0, 8, scale_group, 0)
        pltpu.sync_copy(rows[b], agg.at[dst[b]], add=True)

    stage(gbase, 0)

    def outer(c0, carry):
        for b in range(2):
            c = c0 * 2 + b
            cn = jnp.minimum(c + 1, NCHUNK - 1)
            stage(gbase + cn, 1 - b)
            consume(b)
        return carry

    lax.fori_loop(0, NCHUNK // 2, outer, 0)
    pltpu.make_async_copy(mbig_hbm.at[idx0], rows0, sem0).wait()
    plsc.subcore_barrier()
    row = pl.multiple_of(sid * 624, 8)
    orow = pl.multiple_of(cid * N + sid * 624, 8)
    pltpu.sync_copy(agg.at[pl.ds(row, 624)], out_hbm.at[pl.ds(orow, 624)])

    @pl.when(sid == NS - 1)
    def _tail():
        t0 = pl.multiple_of(16 * 624, 8)
        to = pl.multiple_of(cid * N + 16 * 624, 8)
        pltpu.sync_copy(agg.at[pl.ds(t0, N - 16 * 624)],
                        out_hbm.at[pl.ds(to, N - 16 * 624)])


def _sc_edges(mbig, packed, cgp):
    mesh = plsc.VectorSubcoreMesh(core_axis_name="c", subcore_axis_name="s",
                                  num_cores=NC, num_subcores=NS)
    return pl.kernel(
        _sc_edges_body,
        out_type=jax.ShapeDtypeStruct((NC * N, D), F32),
        mesh=mesh,
        scratch_types=[
            pltpu.VMEM((3 * K,), jnp.int32),
            pltpu.VMEM((3 * K,), jnp.int32),
            pltpu.VMEM((K,), F32),
            pltpu.VMEM((K,), F32),
            pltpu.VMEM((K,), jnp.int32),
            pltpu.VMEM((K,), jnp.int32),
            pltpu.VMEM((K,), jnp.int32),
            pltpu.VMEM((K,), jnp.int32),
            pltpu.VMEM((K, D), F32),
            pltpu.VMEM((K, D), F32),
            pltpu.VMEM_SHARED((N, D), F32),
            pltpu.SemaphoreType.DMA,
            pltpu.SemaphoreType.DMA,
        ],
    )(mbig, packed, cgp)


def _epi_body(p_ref, xsl_ref, i1_ref, i2_ref, f2_ref, tar_ref, zp_ref,
              nr_ref, tg_ref, pa_ref):
    h = p_ref[0, 0] + p_ref[1, 0] + xsl_ref[0]
    nf = jnp.maximum(h, 0.0)
    scat = jnp.where(i2_ref[0] != 0, f2_ref[0] + 1, R + 1)
    iota_j = lax.broadcasted_iota(jnp.int32, (16, NPG), 1)
    rid = lax.broadcasted_iota(jnp.int32, (16, NPG), 0)
    m = (scat == rid) & (rid < R + 1)
    jsel = jnp.max(jnp.where(m, iota_j, -1), axis=1, keepdims=True)
    jt = jnp.min(jnp.where(tar_ref[0] == 1, iota_j[0:1, :], N))
    rr = jnp.max(rid, axis=1, keepdims=True)
    jfin = jnp.where(rr == R + 1, jt, jsel)
    oh = (iota_j == jfin).astype(F32)
    sel = jnp.dot(oh, nf, preferred_element_type=F32,
                  precision=lax.Precision.HIGHEST)
    nr_ref[0] = sel[0:R + 1, :]
    tg = sel[R + 1:R + 2, :]
    tg_ref[0] = tg
    logits = jnp.dot(nf, tg.T, preferred_element_type=F32,
                     precision=lax.Precision.HIGHEST)
    pm = i1_ref[0] == 1
    mx = jnp.max(jnp.where(pm, logits, -1e30))
    ex = jnp.where(pm, jnp.exp(logits - mx), 0.0)
    s = jnp.sum(ex)
    pooled = jnp.sum(nf * ex, axis=0, keepdims=True) / jnp.maximum(s, 1e-30)
    pa_ref[0] = jnp.where(s > 0.0, pooled, zp_ref[...])


def _epilogue(parts, xsl, index1, index2, f2, tar_rel, zero_path):
    p4 = parts.reshape(NC, B, NPG, D)
    xsl3 = xsl.reshape(B, NPG, D)
    i1 = index1.reshape(B, NPG, 1)
    i2 = index2.reshape(B, 1, NPG)
    f2r = f2.reshape(B, 1, NPG)
    tr = tar_rel.reshape(B, 1, NPG)
    nr, tg, pa = pl.pallas_call(
        _epi_body,
        grid=(B,),
        in_specs=[
            pl.BlockSpec((NC, 1, NPG, D), lambda i: (0, i, 0, 0)),
            pl.BlockSpec((1, NPG, D), lambda i: (i, 0, 0)),
            pl.BlockSpec((1, NPG, 1), lambda i: (i, 0, 0)),
            pl.BlockSpec((1, 1, NPG), lambda i: (i, 0, 0)),
            pl.BlockSpec((1, 1, NPG), lambda i: (i, 0, 0)),
            pl.BlockSpec((1, 1, NPG), lambda i: (i, 0, 0)),
            pl.BlockSpec((1, D), lambda i: (0, 0)),
        ],
        out_specs=(
            pl.BlockSpec((1, R + 1, D), lambda i: (i, 0, 0)),
            pl.BlockSpec((1, 1, D), lambda i: (i, 0, 0)),
            pl.BlockSpec((1, 1, D), lambda i: (i, 0, 0)),
        ),
        out_shape=(
            jax.ShapeDtypeStruct((B, R + 1, D), F32),
            jax.ShapeDtypeStruct((B, 1, D), F32),
            jax.ShapeDtypeStruct((B, 1, D), F32),
        ),
    )(p4, xsl3, i1, i2, f2r, tr, zero_path)
    return nr, tg.reshape(B, D), pa.reshape(B, D)


def kernel(x, edge_index, edge_type, cg_norm, index1, index2, f2, tar_rel,
           W_w, b_w, W_w2, b_w2, attn_rel_emb, self_loop_weight, zero_path,
           W_line, b_line):
    AB, C2, V, SLL = _wprep(W_w2, attn_rel_emb, b_w2, W_line,
                            self_loop_weight)
    mbig, xsl = _tables(x, W_w, b_w, AB, C2, V, SLL, b_line, attn_rel_emb)

    pad = EPAD - E
    zi = jnp.zeros((pad,), jnp.int32)
    etp = jnp.concatenate([edge_type, zi])
    srcp = jnp.concatenate([edge_index[0], zi])
    dstp = jnp.concatenate([edge_index[1], zi])
    cgp = jnp.concatenate([cg_norm, jnp.zeros((pad,), F32)])
    packed = jnp.stack(
        [a.reshape(NW * NCHUNK, K) for a in (etp, srcp, dstp)],
        axis=1).reshape(-1)

    parts = _sc_edges(mbig, packed, cgp)
    return _epilogue(parts, xsl, index1, index2, f2, tar_rel, zero_path)
